# Initial kernel scaffold; baseline (speedup 1.0000x reference)
import jax
import jax.numpy as jnp
from jax.experimental import pallas as pl


def kernel(idxes, pe):
    raise NotImplementedError("write your pallas kernel here")


# speedup vs baseline: 3.4502x; 3.4502x over previous
import functools

import jax
import jax.numpy as jnp
from jax import lax
from jax.experimental import pallas as pl
from jax.experimental.pallas import tpu as pltpu
from jax.experimental.pallas import tpu_sc as plsc

B = 4096 * 50
D = 64
NC, NS = 2, 16
NW = NC * NS
B_PER_W = B // NW
CHUNK = 800
NCHUNK = B_PER_W // CHUNK


def _gather_body(idx_hbm, table_hbm, out_hbm, idx_v, rows_v, sem_idx, sem_g, sem_o):
    wid = lax.axis_index("s") * NC + lax.axis_index("c")
    base = wid * B_PER_W
    for c in range(NCHUNK):
        off = base + c * CHUNK
        pltpu.sync_copy(idx_hbm.at[pl.ds(off, CHUNK)], idx_v)
        pltpu.async_copy(table_hbm.at[idx_v], rows_v, sem_g).wait()
        pltpu.sync_copy(rows_v, out_hbm.at[pl.ds(off, CHUNK)])


@jax.jit
def _sc_gather(idx_flat, pe):
    mesh = plsc.VectorSubcoreMesh(core_axis_name="c", subcore_axis_name="s")
    k = pl.kernel(
        _gather_body,
        out_type=jax.ShapeDtypeStruct((B, D), jnp.float32),
        mesh=mesh,
        scratch_types=[
            pltpu.VMEM((CHUNK,), jnp.int32),
            pltpu.VMEM((CHUNK, D), jnp.float32),
            pltpu.SemaphoreType.DMA,
            pltpu.SemaphoreType.DMA,
            pltpu.SemaphoreType.DMA,
        ],
        compiler_params=pltpu.CompilerParams(use_tc_tiling_on_sc=False),
    )
    return k(idx_flat, pe)


def kernel(idxes, pe):
    idx_flat = idxes.reshape(B).astype(jnp.int32)
    out = _sc_gather(idx_flat, pe)
    return out.reshape(idxes.shape + (D,))


# speedup vs baseline: 3.5295x; 1.0230x over previous
import functools

import jax
import jax.numpy as jnp
from jax import lax
from jax.experimental import pallas as pl
from jax.experimental.pallas import tpu as pltpu
from jax.experimental.pallas import tpu_sc as plsc

B = 4096 * 50
D = 64
NC, NS = 2, 16
NW = NC * NS
B_PER_W = B // NW
CHUNK = 800
NCHUNK = B_PER_W // CHUNK


def _gather_body(idx_hbm, table_hbm, out_hbm, idx_v, rows0, rows1,
                 sem_g0, sem_g1, sem_o0, sem_o1):
    wid = lax.axis_index("s") * NC + lax.axis_index("c")
    base = wid * B_PER_W
    rows = (rows0, rows1)
    sem_g = (sem_g0, sem_g1)
    sem_o = (sem_o0, sem_o1)
    pltpu.sync_copy(idx_hbm.at[pl.ds(base, B_PER_W)], idx_v)
    g = [None, None]
    o = [None, None]
    for c in range(NCHUNK):
        b = c % 2
        if c >= 2:
            o[b].wait()
        g[b] = pltpu.async_copy(
            table_hbm.at[idx_v.at[pl.ds(c * CHUNK, CHUNK)]], rows[b], sem_g[b])
        if c >= 1:
            pb = (c - 1) % 2
            g[pb].wait()
            o[pb] = pltpu.async_copy(
                rows[pb], out_hbm.at[pl.ds(base + (c - 1) * CHUNK, CHUNK)],
                sem_o[pb])
    lb = (NCHUNK - 1) % 2
    g[lb].wait()
    o[lb] = pltpu.async_copy(
        rows[lb], out_hbm.at[pl.ds(base + (NCHUNK - 1) * CHUNK, CHUNK)],
        sem_o[lb])
    o[1 - lb].wait()
    o[lb].wait()


@jax.jit
def _sc_gather(idx_flat, pe):
    mesh = plsc.VectorSubcoreMesh(core_axis_name="c", subcore_axis_name="s")
    k = pl.kernel(
        _gather_body,
        out_type=jax.ShapeDtypeStruct((B, D), jnp.float32),
        mesh=mesh,
        scratch_types=[
            pltpu.VMEM((B_PER_W,), jnp.int32),
            pltpu.VMEM((CHUNK, D), jnp.float32),
            pltpu.VMEM((CHUNK, D), jnp.float32),
            pltpu.SemaphoreType.DMA,
            pltpu.SemaphoreType.DMA,
            pltpu.SemaphoreType.DMA,
            pltpu.SemaphoreType.DMA,
        ],
        compiler_params=pltpu.CompilerParams(use_tc_tiling_on_sc=False),
    )
    return k(idx_flat, pe)


def kernel(idxes, pe):
    idx_flat = idxes.reshape(B).astype(jnp.int32)
    out = _sc_gather(idx_flat, pe)
    return out.reshape(idxes.shape + (D,))


# speedup vs baseline: 4.3168x; 1.2231x over previous
import jax
import jax.numpy as jnp
from jax import lax
from jax.experimental import pallas as pl
from jax.experimental.pallas import tpu as pltpu
from jax.experimental.pallas import tpu_sc as plsc

N_I = 4096
N_J = 50
N_JP = 56
B = N_I * N_J
D = 64
DP = 128
NC, NS = 2, 16
NW = NC * NS
I_PER_W = N_I // NW
BLKS = 8
CHUNK = BLKS * N_J
NCHUNK = I_PER_W // BLKS
B_PER_W = I_PER_W * N_J
ROWS_V = CHUNK + (N_JP - N_J)


def _gather_body(idx_hbm, table_hbm, out_hbm, idx_v, rows0, rows1,
                 sem_g0, sem_g1, sem_o0, sem_o1):
    wid = lax.axis_index("s") * NC + lax.axis_index("c")
    base = wid * B_PER_W
    i_base = wid * I_PER_W
    rows = (rows0, rows1)
    sem_g = (sem_g0, sem_g1)
    sem_o = (sem_o0, sem_o1)
    pltpu.sync_copy(idx_hbm.at[pl.ds(base, B_PER_W)], idx_v)
    g = [None, None]
    o = [[], []]

    def writeback(c, buf):
        i0 = i_base + c * BLKS
        for k in range(BLKS):
            o[buf].append(pltpu.async_copy(
                rows[buf].at[pl.ds(k * N_J, N_JP)],
                out_hbm.at[i0 + k], sem_o[buf]))

    for c in range(NCHUNK):
        b = c % 2
        if c >= 2:
            for cp in o[b]:
                cp.wait()
            o[b] = []
        g[b] = pltpu.async_copy(
            table_hbm.at[idx_v.at[pl.ds(c * CHUNK, CHUNK)]],
            rows[b].at[pl.ds(0, CHUNK)], sem_g[b])
        if c >= 1:
            g[1 - b].wait()
            writeback(c - 1, 1 - b)
    lb = (NCHUNK - 1) % 2
    g[lb].wait()
    writeback(NCHUNK - 1, lb)
    for buf in (0, 1):
        for cp in o[buf]:
            cp.wait()


@jax.jit
def _sc_gather(idx_flat, pe_pad):
    mesh = plsc.VectorSubcoreMesh(core_axis_name="c", subcore_axis_name="s")
    k = pl.kernel(
        _gather_body,
        out_type=jax.ShapeDtypeStruct((N_I, N_JP, DP), jnp.float32),
        mesh=mesh,
        scratch_types=[
            pltpu.VMEM((B_PER_W,), jnp.int32),
            pltpu.VMEM((ROWS_V, DP), jnp.float32),
            pltpu.VMEM((ROWS_V, DP), jnp.float32),
            pltpu.SemaphoreType.DMA,
            pltpu.SemaphoreType.DMA,
            pltpu.SemaphoreType.DMA,
            pltpu.SemaphoreType.DMA,
        ],
    )
    return k(idx_flat, pe_pad)


def kernel(idxes, pe):
    idx_flat = idxes.reshape(B).astype(jnp.int32)
    pe_pad = jnp.pad(pe, ((0, 0), (0, DP - D)))
    out56 = _sc_gather(idx_flat, pe_pad)
    return out56[:, :N_J, :D]


# speedup vs baseline: 4.3194x; 1.0006x over previous
import jax
import jax.numpy as jnp
from jax import lax
from jax.experimental import pallas as pl
from jax.experimental.pallas import tpu as pltpu
from jax.experimental.pallas import tpu_sc as plsc

N_I = 4096
N_J = 50
D = 64
DP = 128
NC, NS = 2, 16
NW = NC * NS
I_PER_W = N_I // NW
BLKS = 8
CHUNK = BLKS * N_J
NCHUNK = I_PER_W // BLKS


def _gather_body(idx_hbm, table_hbm, out_hbm, idx_v, rows0, rows1,
                 sem_i, sem_g0, sem_g1, sem_o0, sem_o1):
    wid = lax.axis_index("s") * NC + lax.axis_index("c")
    i_base = wid * I_PER_W
    rows = (rows0, rows1)
    sem_g = (sem_g0, sem_g1)
    sem_o = (sem_o0, sem_o1)
    pltpu.sync_copy(idx_hbm.at[pl.ds(i_base, I_PER_W)], idx_v)
    g = [[], []]
    o = [[], []]

    def gather(c, buf):
        for k in range(BLKS):
            g[buf].append(pltpu.async_copy(
                table_hbm.at[idx_v.at[c * BLKS + k]],
                rows[buf].at[pl.ds(k * N_J, N_J)], sem_g[buf]))

    def writeback(c, buf):
        i0 = i_base + c * BLKS
        for k in range(BLKS):
            o[buf].append(pltpu.async_copy(
                rows[buf].at[pl.ds(k * N_J, 56)],
                out_hbm.at[i0 + k], sem_o[buf]))

    for c in range(NCHUNK):
        b = c % 2
        if c >= 2:
            for cp in o[b]:
                cp.wait()
            o[b] = []
        gather(c, b)
        if c >= 1:
            for cp in g[1 - b]:
                cp.wait()
            g[1 - b] = []
            writeback(c - 1, 1 - b)
    lb = (NCHUNK - 1) % 2
    for cp in g[lb]:
        cp.wait()
    writeback(NCHUNK - 1, lb)
    for buf in (0, 1):
        for cp in o[buf]:
            cp.wait()


@jax.jit
def _sc_gather(idxes, pe_pad):
    mesh = plsc.VectorSubcoreMesh(core_axis_name="c", subcore_axis_name="s")
    k = pl.kernel(
        _gather_body,
        out_type=jax.ShapeDtypeStruct((N_I, 56, DP), jnp.float32),
        mesh=mesh,
        scratch_types=[
            pltpu.VMEM((I_PER_W, N_J), jnp.int32),
            pltpu.VMEM((CHUNK + 6, DP), jnp.float32),
            pltpu.VMEM((CHUNK + 6, DP), jnp.float32),
            pltpu.SemaphoreType.DMA,
            pltpu.SemaphoreType.DMA,
            pltpu.SemaphoreType.DMA,
            pltpu.SemaphoreType.DMA,
            pltpu.SemaphoreType.DMA,
        ],
    )
    return k(idxes, pe_pad)


def kernel(idxes, pe):
    pe_pad = jnp.pad(pe, ((0, 0), (0, DP - D)))
    out56 = _sc_gather(idxes.astype(jnp.int32), pe_pad)
    return out56[:, :N_J, :D]
